# baseline (speedup 1.0000x reference)
import functools

import jax
import jax.numpy as jnp
from jax import lax
from jax.experimental import pallas as pl
from jax.experimental.pallas import tpu as pltpu
from jax.experimental.pallas import tpu_sc as plsc

N_NODES = 10000
N_EDGES = 320000
IN_DIM = 128
HIDDEN_DIM = 128
OUT_DIM = 64
NLAYERS = 2
HALF = OUT_DIM // 2

NC = 2
NS = 16
EDGES_PER_T = N_EDGES // NS
CHUNK = 128
NFULL = EDGES_PER_T // CHUNK
TAIL = EDGES_PER_T - NFULL * CHUNK
ROWS_PER_TILE = N_NODES // NS
NBUF = 4
PREF = 3


def _mlp_body(x_ref, w1_ref, w2_ref, lo_ref, hi_ref):
    h = jnp.dot(x_ref[...], w1_ref[...], preferred_element_type=jnp.float32)
    h = jnp.maximum(h, 0.0)
    o = jnp.dot(h, w2_ref[...], preferred_element_type=jnp.float32)
    lo_ref[...] = o[:, :HALF]
    hi_ref[...] = o[:, HALF:]


_mlp = pl.pallas_call(
    _mlp_body,
    grid=(10,),
    in_specs=[
        pl.BlockSpec((N_NODES // 10, IN_DIM), lambda i: (i, 0)),
        pl.BlockSpec((IN_DIM, HIDDEN_DIM), lambda i: (0, 0)),
        pl.BlockSpec((HIDDEN_DIM, OUT_DIM), lambda i: (0, 0)),
    ],
    out_specs=[
        pl.BlockSpec((N_NODES // 10, HALF), lambda i: (i, 0)),
        pl.BlockSpec((N_NODES // 10, HALF), lambda i: (i, 0)),
    ],
    out_shape=[
        jax.ShapeDtypeStruct((N_NODES, HALF), jnp.float32),
        jax.ShapeDtypeStruct((N_NODES, HALF), jnp.float32),
    ],
)


def _cat_body(a_ref, b_ref, o_ref):
    o_ref[...] = jnp.concatenate([a_ref[...], b_ref[...]], axis=1)


_cat_halves = pl.pallas_call(
    _cat_body,
    grid=(10,),
    in_specs=[
        pl.BlockSpec((N_NODES // 10, HALF), lambda i: (i, 0)),
        pl.BlockSpec((N_NODES // 10, HALF), lambda i: (i, 0)),
    ],
    out_specs=pl.BlockSpec((N_NODES // 10, OUT_DIM), lambda i: (i, 0)),
    out_shape=jax.ShapeDtypeStruct((N_NODES, OUT_DIM), jnp.float32),
)


_sc_mesh = plsc.VectorSubcoreMesh(
    core_axis_name="c", subcore_axis_name="s", num_cores=NC, num_subcores=NS
)


@functools.partial(
    pl.kernel,
    out_type=[
        jax.ShapeDtypeStruct((N_NODES, HALF), jnp.float32),
        jax.ShapeDtypeStruct((N_NODES, HALF), jnp.float32),
    ],
    mesh=_sc_mesh,
    compiler_params=pltpu.CompilerParams(use_tc_tiling_on_sc=False),
    scratch_types=[
        pltpu.VMEM((EDGES_PER_T,), jnp.int32),
        pltpu.VMEM((EDGES_PER_T,), jnp.int32),
        pltpu.VMEM((NBUF + 1, CHUNK, HALF), jnp.float32),
        pltpu.VMEM_SHARED((N_NODES, HALF), jnp.float32),
        [pltpu.SemaphoreType.DMA] * (NBUF + 1),
    ],
)
def _propagate(hlo_hbm, hhi_hbm, edge_hbm, outlo_hbm, outhi_hbm,
               src_v, dst_v, rows_v, acc_sh, gsems):
    cid = lax.axis_index("c")
    sid = lax.axis_index("s")

    zeros16 = jnp.zeros((16,), jnp.float32)

    def _zero_row(r, _):
        for b in range(NBUF + 1):
            for l in range(HALF // 16):
                rows_v[b, r, pl.ds(l * 16, 16)] = zeros16
        return 0

    lax.fori_loop(0, CHUNK, _zero_row, 0)
    for k in range(5):
        pltpu.sync_copy(
            rows_v.at[k % NBUF, pl.ds(0, 125)],
            acc_sh.at[pl.ds(sid * ROWS_PER_TILE + k * 125, 125)],
        )
    plsc.subcore_barrier()

    pltpu.sync_copy(edge_hbm.at[0, pl.ds(sid * EDGES_PER_T, EDGES_PER_T)], src_v)
    pltpu.sync_copy(edge_hbm.at[1, pl.ds(sid * EDGES_PER_T, EDGES_PER_T)], dst_v)

    def _gather_start(ci, b):
        idx = src_v.at[pl.ds(ci * CHUNK, CHUNK)]

        @pl.when(cid == 0)
        def _():
            pltpu.make_async_copy(hlo_hbm.at[idx], rows_v.at[b], gsems[b]).start()

        @pl.when(cid == 1)
        def _():
            pltpu.make_async_copy(hhi_hbm.at[idx], rows_v.at[b], gsems[b]).start()

    def _gather_wait(ci, b):
        pltpu.make_async_copy(
            hlo_hbm.at[src_v.at[pl.ds(ci * CHUNK, CHUNK)]], rows_v.at[b],
            gsems[b]).wait()

    tail_idx = src_v.at[pl.ds(NFULL * CHUNK, TAIL)]

    @pl.when(cid == 0)
    def _():
        pltpu.make_async_copy(hlo_hbm.at[tail_idx],
                              rows_v.at[NBUF, pl.ds(0, TAIL)], gsems[NBUF]).start()

    @pl.when(cid == 1)
    def _():
        pltpu.make_async_copy(hhi_hbm.at[tail_idx],
                              rows_v.at[NBUF, pl.ds(0, TAIL)], gsems[NBUF]).start()

    for b in range(PREF):
        _gather_start(b, b)

    def _chunk_body(ci, b):
        _gather_wait(ci, b)
        pltpu.sync_copy(rows_v.at[b],
                        acc_sh.at[dst_v.at[pl.ds(ci * CHUNK, CHUNK)]],
                        add=True)

        @pl.when(ci + PREF < NFULL)
        def _():
            _gather_start(ci + PREF, (b + PREF) % NBUF)

    def _step(k, _):
        i0 = k * NBUF
        for b in range(NBUF):
            _chunk_body(i0 + b, b)
        return 0

    lax.fori_loop(0, NFULL // NBUF, _step, 0)

    pltpu.make_async_copy(hlo_hbm.at[tail_idx],
                          rows_v.at[NBUF, pl.ds(0, TAIL)], gsems[NBUF]).wait()
    pltpu.sync_copy(rows_v.at[NBUF, pl.ds(0, TAIL)],
                    acc_sh.at[dst_v.at[pl.ds(NFULL * CHUNK, TAIL)]],
                    add=True)
    plsc.subcore_barrier()

    stripe = acc_sh.at[pl.ds(sid * ROWS_PER_TILE, ROWS_PER_TILE)]

    @pl.when(cid == 0)
    def _():
        pltpu.sync_copy(stripe,
                        outlo_hbm.at[pl.ds(sid * ROWS_PER_TILE, ROWS_PER_TILE)])

    @pl.when(cid == 1)
    def _():
        pltpu.sync_copy(stripe,
                        outhi_hbm.at[pl.ds(sid * ROWS_PER_TILE, ROWS_PER_TILE)])


@jax.jit
def kernel(features, edge_index, W1, W2):
    h_lo, h_hi = _mlp(features, W1, W2)
    edges = edge_index.astype(jnp.int32)
    for _ in range(NLAYERS):
        h_lo, h_hi = _propagate(h_lo, h_hi, edges)
    return _cat_halves(h_lo, h_hi)


# speedup vs baseline: 1.0897x; 1.0897x over previous
import functools

import jax
import jax.numpy as jnp
from jax import lax
from jax.experimental import pallas as pl
from jax.experimental.pallas import tpu as pltpu
from jax.experimental.pallas import tpu_sc as plsc

N_NODES = 10000
N_EDGES = 320000
IN_DIM = 128
HIDDEN_DIM = 128
OUT_DIM = 64
NLAYERS = 2
HALF = OUT_DIM // 2

NC = 2
NS = 16
EDGES_PER_T = N_EDGES // NS
CHUNK = 128
NFULL = EDGES_PER_T // CHUNK
TAIL = EDGES_PER_T - NFULL * CHUNK
ROWS_PER_TILE = N_NODES // NS
NBUF = 4
PREF = 3


def _mlp_body(x_ref, w1_ref, w2_ref, lo_ref, hi_ref):
    h = jnp.dot(x_ref[...], w1_ref[...], preferred_element_type=jnp.float32)
    h = jnp.maximum(h, 0.0)
    o = jnp.dot(h, w2_ref[...], preferred_element_type=jnp.float32)
    lo_ref[...] = o[:, :HALF]
    hi_ref[...] = o[:, HALF:]


_mlp = pl.pallas_call(
    _mlp_body,
    grid=(10,),
    in_specs=[
        pl.BlockSpec((N_NODES // 10, IN_DIM), lambda i: (i, 0)),
        pl.BlockSpec((IN_DIM, HIDDEN_DIM), lambda i: (0, 0)),
        pl.BlockSpec((HIDDEN_DIM, OUT_DIM), lambda i: (0, 0)),
    ],
    out_specs=[
        pl.BlockSpec((N_NODES // 10, HALF), lambda i: (i, 0)),
        pl.BlockSpec((N_NODES // 10, HALF), lambda i: (i, 0)),
    ],
    out_shape=[
        jax.ShapeDtypeStruct((N_NODES, HALF), jnp.float32),
        jax.ShapeDtypeStruct((N_NODES, HALF), jnp.float32),
    ],
)


_sc_mesh = plsc.VectorSubcoreMesh(
    core_axis_name="c", subcore_axis_name="s", num_cores=NC, num_subcores=NS
)


def _propagate_body(interleave, hlo_hbm, hhi_hbm, edge_hbm, out_refs,
                    src_v, dst_v, rows_v, acc_sh, gsems):
    cid = lax.axis_index("c")
    sid = lax.axis_index("s")

    zeros16 = jnp.zeros((16,), jnp.float32)

    def _zero_row(r, _):
        for b in range(NBUF + 1):
            for l in range(HALF // 16):
                rows_v[b, r, pl.ds(l * 16, 16)] = zeros16
        return 0

    lax.fori_loop(0, CHUNK, _zero_row, 0)
    for k in range(5):
        pltpu.sync_copy(
            rows_v.at[k % NBUF, pl.ds(0, 125)],
            acc_sh.at[pl.ds(sid * ROWS_PER_TILE + k * 125, 125)],
        )
    plsc.subcore_barrier()

    pltpu.sync_copy(edge_hbm.at[pl.ds(sid * EDGES_PER_T, EDGES_PER_T)], src_v)
    pltpu.sync_copy(
        edge_hbm.at[pl.ds(N_EDGES + sid * EDGES_PER_T, EDGES_PER_T)], dst_v)

    def _gather_start(ci, b):
        idx = src_v.at[pl.ds(ci * CHUNK, CHUNK)]

        @pl.when(cid == 0)
        def _():
            pltpu.make_async_copy(hlo_hbm.at[idx], rows_v.at[b], gsems[b]).start()

        @pl.when(cid == 1)
        def _():
            pltpu.make_async_copy(hhi_hbm.at[idx], rows_v.at[b], gsems[b]).start()

    def _gather_wait(ci, b):
        pltpu.make_async_copy(
            hlo_hbm.at[src_v.at[pl.ds(ci * CHUNK, CHUNK)]], rows_v.at[b],
            gsems[b]).wait()

    tail_idx = src_v.at[pl.ds(NFULL * CHUNK, TAIL)]

    @pl.when(cid == 0)
    def _():
        pltpu.make_async_copy(hlo_hbm.at[tail_idx],
                              rows_v.at[NBUF, pl.ds(0, TAIL)], gsems[NBUF]).start()

    @pl.when(cid == 1)
    def _():
        pltpu.make_async_copy(hhi_hbm.at[tail_idx],
                              rows_v.at[NBUF, pl.ds(0, TAIL)], gsems[NBUF]).start()

    for b in range(PREF):
        _gather_start(b, b)

    def _chunk_body(ci, b):
        _gather_wait(ci, b)
        pltpu.sync_copy(rows_v.at[b],
                        acc_sh.at[dst_v.at[pl.ds(ci * CHUNK, CHUNK)]],
                        add=True)

        @pl.when(ci + PREF < NFULL)
        def _():
            _gather_start(ci + PREF, (b + PREF) % NBUF)

    def _step(k, _):
        i0 = k * NBUF
        for b in range(NBUF):
            _chunk_body(i0 + b, b)
        return 0

    lax.fori_loop(0, NFULL // NBUF, _step, 0)

    pltpu.make_async_copy(hlo_hbm.at[tail_idx],
                          rows_v.at[NBUF, pl.ds(0, TAIL)], gsems[NBUF]).wait()
    pltpu.sync_copy(rows_v.at[NBUF, pl.ds(0, TAIL)],
                    acc_sh.at[dst_v.at[pl.ds(NFULL * CHUNK, TAIL)]],
                    add=True)
    plsc.subcore_barrier()

    stripe = acc_sh.at[pl.ds(sid * ROWS_PER_TILE, ROWS_PER_TILE)]
    if interleave:
        out_hbm, = out_refs

        @pl.when(cid == 0)
        def _():
            pltpu.sync_copy(stripe,
                            out_hbm.at[pl.ds(sid * ROWS_PER_TILE, ROWS_PER_TILE),
                                       pl.ds(0, HALF)])

        @pl.when(cid == 1)
        def _():
            pltpu.sync_copy(stripe,
                            out_hbm.at[pl.ds(sid * ROWS_PER_TILE, ROWS_PER_TILE),
                                       pl.ds(HALF, HALF)])
    else:
        outlo_hbm, outhi_hbm = out_refs

        @pl.when(cid == 0)
        def _():
            pltpu.sync_copy(
                stripe, outlo_hbm.at[pl.ds(sid * ROWS_PER_TILE, ROWS_PER_TILE)])

        @pl.when(cid == 1)
        def _():
            pltpu.sync_copy(
                stripe, outhi_hbm.at[pl.ds(sid * ROWS_PER_TILE, ROWS_PER_TILE)])


_prop_out_split = [
    jax.ShapeDtypeStruct((N_NODES, HALF), jnp.float32),
    jax.ShapeDtypeStruct((N_NODES, HALF), jnp.float32),
]
_prop_out_full = [jax.ShapeDtypeStruct((N_NODES, OUT_DIM), jnp.float32)]
_prop_scratch = [
    pltpu.VMEM((EDGES_PER_T,), jnp.int32),
    pltpu.VMEM((EDGES_PER_T,), jnp.int32),
    pltpu.VMEM((NBUF + 1, CHUNK, HALF), jnp.float32),
    pltpu.VMEM_SHARED((N_NODES, HALF), jnp.float32),
    [pltpu.SemaphoreType.DMA] * (NBUF + 1),
]


@functools.partial(pl.kernel, out_type=_prop_out_split, mesh=_sc_mesh,
                   compiler_params=pltpu.CompilerParams(use_tc_tiling_on_sc=False),
                   scratch_types=_prop_scratch)
def _propagate_split(hlo, hhi, edge, outlo, outhi, src_v, dst_v, rows_v,
                     acc_sh, gsems):
    _propagate_body(False, hlo, hhi, edge, (outlo, outhi),
                    src_v, dst_v, rows_v, acc_sh, gsems)


@functools.partial(pl.kernel, out_type=_prop_out_full, mesh=_sc_mesh,
                   compiler_params=pltpu.CompilerParams(use_tc_tiling_on_sc=False),
                   scratch_types=_prop_scratch)
def _propagate_full(hlo, hhi, edge, out, src_v, dst_v, rows_v, acc_sh, gsems):
    _propagate_body(True, hlo, hhi, edge, (out,),
                    src_v, dst_v, rows_v, acc_sh, gsems)


@jax.jit
def kernel(features, edge_index, W1, W2):
    h_lo, h_hi = _mlp(features, W1, W2)
    edges = edge_index.astype(jnp.int32).reshape(-1)
    for _ in range(NLAYERS - 1):
        h_lo, h_hi = _propagate_split(h_lo, h_hi, edges)
    out, = _propagate_full(h_lo, h_hi, edges)
    return out


# speedup vs baseline: 1.1819x; 1.0846x over previous
import functools

import jax
import jax.numpy as jnp
from jax import lax
from jax.experimental import pallas as pl
from jax.experimental.pallas import tpu as pltpu
from jax.experimental.pallas import tpu_sc as plsc

N_NODES = 10000
N_EDGES = 320000
IN_DIM = 128
HIDDEN_DIM = 128
OUT_DIM = 64
NLAYERS = 2
HALF = OUT_DIM // 2

NC = 2
NS = 16
EDGES_PER_T = N_EDGES // NS
CHUNK = 128
NFULL = EDGES_PER_T // CHUNK
TAIL = EDGES_PER_T - NFULL * CHUNK
ROWS_PER_TILE = N_NODES // NS
NBUF = 8
PREF = 4


def _mlp_body(x_ref, w1_ref, w2_ref, lo_ref, hi_ref):
    h = jnp.dot(x_ref[...], w1_ref[...], preferred_element_type=jnp.float32)
    h = jnp.maximum(h, 0.0)
    o = jnp.dot(h, w2_ref[...], preferred_element_type=jnp.float32)
    lo_ref[...] = o[:, :HALF]
    hi_ref[...] = o[:, HALF:]


_mlp = pl.pallas_call(
    _mlp_body,
    grid=(10,),
    in_specs=[
        pl.BlockSpec((N_NODES // 10, IN_DIM), lambda i: (i, 0)),
        pl.BlockSpec((IN_DIM, HIDDEN_DIM), lambda i: (0, 0)),
        pl.BlockSpec((HIDDEN_DIM, OUT_DIM), lambda i: (0, 0)),
    ],
    out_specs=[
        pl.BlockSpec((N_NODES // 10, HALF), lambda i: (i, 0)),
        pl.BlockSpec((N_NODES // 10, HALF), lambda i: (i, 0)),
    ],
    out_shape=[
        jax.ShapeDtypeStruct((N_NODES, HALF), jnp.float32),
        jax.ShapeDtypeStruct((N_NODES, HALF), jnp.float32),
    ],
)


_sc_mesh = plsc.VectorSubcoreMesh(
    core_axis_name="c", subcore_axis_name="s", num_cores=NC, num_subcores=NS
)


def _propagate_body(interleave, hlo_hbm, hhi_hbm, edge_hbm, out_refs,
                    src_v, dst_v, rows_v, acc_sh, gsems, ssems):
    cid = lax.axis_index("c")
    sid = lax.axis_index("s")

    zeros16 = jnp.zeros((16,), jnp.float32)

    def _zero_row(r, _):
        for b in range(NBUF + 1):
            for l in range(HALF // 16):
                rows_v[b, r, pl.ds(l * 16, 16)] = zeros16
        return 0

    lax.fori_loop(0, CHUNK, _zero_row, 0)
    for k in range(5):
        pltpu.sync_copy(
            rows_v.at[k % NBUF, pl.ds(0, 125)],
            acc_sh.at[pl.ds(sid * ROWS_PER_TILE + k * 125, 125)],
        )
    plsc.subcore_barrier()

    pltpu.sync_copy(edge_hbm.at[pl.ds(sid * EDGES_PER_T, EDGES_PER_T)], src_v)
    pltpu.sync_copy(
        edge_hbm.at[pl.ds(N_EDGES + sid * EDGES_PER_T, EDGES_PER_T)], dst_v)

    def _gather_start(ci, b):
        idx = src_v.at[pl.ds(ci * CHUNK, CHUNK)]

        @pl.when(cid == 0)
        def _():
            pltpu.make_async_copy(hlo_hbm.at[idx], rows_v.at[b], gsems[b]).start()

        @pl.when(cid == 1)
        def _():
            pltpu.make_async_copy(hhi_hbm.at[idx], rows_v.at[b], gsems[b]).start()

    def _gather_wait(ci, b):
        pltpu.make_async_copy(
            hlo_hbm.at[src_v.at[pl.ds(ci * CHUNK, CHUNK)]], rows_v.at[b],
            gsems[b]).wait()

    tail_idx = src_v.at[pl.ds(NFULL * CHUNK, TAIL)]

    @pl.when(cid == 0)
    def _():
        pltpu.make_async_copy(hlo_hbm.at[tail_idx],
                              rows_v.at[NBUF, pl.ds(0, TAIL)], gsems[NBUF]).start()

    @pl.when(cid == 1)
    def _():
        pltpu.make_async_copy(hhi_hbm.at[tail_idx],
                              rows_v.at[NBUF, pl.ds(0, TAIL)], gsems[NBUF]).start()

    for b in range(PREF):
        _gather_start(b, b)

    def _scatter_start(ci, b):
        pltpu.async_copy(rows_v.at[b],
                         acc_sh.at[dst_v.at[pl.ds(ci * CHUNK, CHUNK)]],
                         ssems[b], add=True)

    def _scatter_wait(ci, b):
        pltpu.make_async_copy(rows_v.at[b],
                              acc_sh.at[dst_v.at[pl.ds(ci * CHUNK, CHUNK)]],
                              ssems[b]).wait()

    def _chunk_body(ci, b):
        _gather_wait(ci, b)
        _scatter_start(ci, b)
        bn = (b + PREF) % NBUF

        @pl.when(ci + PREF < NFULL)
        def _():
            @pl.when(ci >= PREF)
            def _():
                _scatter_wait(ci - PREF, bn)

            _gather_start(ci + PREF, bn)

    def _step(k, _):
        i0 = k * NBUF
        for b in range(NBUF):
            _chunk_body(i0 + b, b)
        return 0

    lax.fori_loop(0, NFULL // NBUF, _step, 0)
    for ci in range((NFULL // NBUF) * NBUF, NFULL):
        _gather_wait(ci, ci % NBUF)
        _scatter_start(ci, ci % NBUF)

    for j in range(NBUF):
        ci = NFULL - NBUF + j
        _scatter_wait(ci, ci % NBUF)

    pltpu.make_async_copy(hlo_hbm.at[tail_idx],
                          rows_v.at[NBUF, pl.ds(0, TAIL)], gsems[NBUF]).wait()
    pltpu.sync_copy(rows_v.at[NBUF, pl.ds(0, TAIL)],
                    acc_sh.at[dst_v.at[pl.ds(NFULL * CHUNK, TAIL)]],
                    add=True)
    plsc.subcore_barrier()

    stripe = acc_sh.at[pl.ds(sid * ROWS_PER_TILE, ROWS_PER_TILE)]
    if interleave:
        out_hbm, = out_refs

        @pl.when(cid == 0)
        def _():
            pltpu.sync_copy(stripe,
                            out_hbm.at[pl.ds(sid * ROWS_PER_TILE, ROWS_PER_TILE),
                                       pl.ds(0, HALF)])

        @pl.when(cid == 1)
        def _():
            pltpu.sync_copy(stripe,
                            out_hbm.at[pl.ds(sid * ROWS_PER_TILE, ROWS_PER_TILE),
                                       pl.ds(HALF, HALF)])
    else:
        outlo_hbm, outhi_hbm = out_refs

        @pl.when(cid == 0)
        def _():
            pltpu.sync_copy(
                stripe, outlo_hbm.at[pl.ds(sid * ROWS_PER_TILE, ROWS_PER_TILE)])

        @pl.when(cid == 1)
        def _():
            pltpu.sync_copy(
                stripe, outhi_hbm.at[pl.ds(sid * ROWS_PER_TILE, ROWS_PER_TILE)])


_prop_out_split = [
    jax.ShapeDtypeStruct((N_NODES, HALF), jnp.float32),
    jax.ShapeDtypeStruct((N_NODES, HALF), jnp.float32),
]
_prop_out_full = [jax.ShapeDtypeStruct((N_NODES, OUT_DIM), jnp.float32)]
_prop_scratch = [
    pltpu.VMEM((EDGES_PER_T,), jnp.int32),
    pltpu.VMEM((EDGES_PER_T,), jnp.int32),
    pltpu.VMEM((NBUF + 1, CHUNK, HALF), jnp.float32),
    pltpu.VMEM_SHARED((N_NODES, HALF), jnp.float32),
    [pltpu.SemaphoreType.DMA] * (NBUF + 1),
    [pltpu.SemaphoreType.DMA] * NBUF,
]


@functools.partial(pl.kernel, out_type=_prop_out_split, mesh=_sc_mesh,
                   compiler_params=pltpu.CompilerParams(use_tc_tiling_on_sc=False),
                   scratch_types=_prop_scratch)
def _propagate_split(hlo, hhi, edge, outlo, outhi, src_v, dst_v, rows_v,
                     acc_sh, gsems, ssems):
    _propagate_body(False, hlo, hhi, edge, (outlo, outhi),
                    src_v, dst_v, rows_v, acc_sh, gsems, ssems)


@functools.partial(pl.kernel, out_type=_prop_out_full, mesh=_sc_mesh,
                   compiler_params=pltpu.CompilerParams(use_tc_tiling_on_sc=False),
                   scratch_types=_prop_scratch)
def _propagate_full(hlo, hhi, edge, out, src_v, dst_v, rows_v, acc_sh,
                    gsems, ssems):
    _propagate_body(True, hlo, hhi, edge, (out,),
                    src_v, dst_v, rows_v, acc_sh, gsems, ssems)


@jax.jit
def kernel(features, edge_index, W1, W2):
    h_lo, h_hi = _mlp(features, W1, W2)
    edges = edge_index.astype(jnp.int32).reshape(-1)
    for _ in range(NLAYERS - 1):
        h_lo, h_hi = _propagate_split(h_lo, h_hi, edges)
    out, = _propagate_full(h_lo, h_hi, edges)
    return out


# speedup vs baseline: 1.2036x; 1.0184x over previous
import functools

import jax
import jax.numpy as jnp
from jax import lax
from jax.experimental import pallas as pl
from jax.experimental.pallas import tpu as pltpu
from jax.experimental.pallas import tpu_sc as plsc

N_NODES = 10000
N_EDGES = 320000
IN_DIM = 128
HIDDEN_DIM = 128
OUT_DIM = 64
NLAYERS = 2
HALF = OUT_DIM // 2

NC = 2
NS = 16
EDGES_PER_T = N_EDGES // NS
CHUNK = 128
NFULL = EDGES_PER_T // CHUNK
TAIL = EDGES_PER_T - NFULL * CHUNK
ROWS_PER_TILE = N_NODES // NS
NBUF = 8
PREF = 4


def _mlp_body(x_ref, w1_ref, w2_ref, lo_ref, hi_ref):
    h = jnp.dot(x_ref[...], w1_ref[...], preferred_element_type=jnp.float32)
    h = jnp.maximum(h, 0.0)
    o = jnp.dot(h, w2_ref[...], preferred_element_type=jnp.float32)
    lo_ref[...] = o[:, :HALF]
    hi_ref[...] = o[:, HALF:]


_MLP_GRID = 5
_mlp = pl.pallas_call(
    _mlp_body,
    grid=(_MLP_GRID,),
    in_specs=[
        pl.BlockSpec((N_NODES // _MLP_GRID, IN_DIM), lambda i: (i, 0)),
        pl.BlockSpec((IN_DIM, HIDDEN_DIM), lambda i: (0, 0)),
        pl.BlockSpec((HIDDEN_DIM, OUT_DIM), lambda i: (0, 0)),
    ],
    out_specs=[
        pl.BlockSpec((N_NODES // _MLP_GRID, HALF), lambda i: (i, 0)),
        pl.BlockSpec((N_NODES // _MLP_GRID, HALF), lambda i: (i, 0)),
    ],
    out_shape=[
        jax.ShapeDtypeStruct((N_NODES, HALF), jnp.float32),
        jax.ShapeDtypeStruct((N_NODES, HALF), jnp.float32),
    ],
)


_sc_mesh = plsc.VectorSubcoreMesh(
    core_axis_name="c", subcore_axis_name="s", num_cores=NC, num_subcores=NS
)


def _propagate_body(interleave, hlo_hbm, hhi_hbm, edge_hbm, out_refs,
                    src_v, dst_v, rows_v, acc_sh, gsems, ssems):
    cid = lax.axis_index("c")
    sid = lax.axis_index("s")

    zeros16 = jnp.zeros((16,), jnp.float32)

    def _zero_row(r, _):
        for b in range(NBUF + 1):
            for l in range(HALF // 16):
                rows_v[b, r, pl.ds(l * 16, 16)] = zeros16
        return 0

    lax.fori_loop(0, CHUNK, _zero_row, 0)
    for k in range(5):
        pltpu.sync_copy(
            rows_v.at[k % NBUF, pl.ds(0, 125)],
            acc_sh.at[pl.ds(sid * ROWS_PER_TILE + k * 125, 125)],
        )
    plsc.subcore_barrier()

    pltpu.sync_copy(edge_hbm.at[pl.ds(sid * EDGES_PER_T, EDGES_PER_T)], src_v)
    pltpu.sync_copy(
        edge_hbm.at[pl.ds(N_EDGES + sid * EDGES_PER_T, EDGES_PER_T)], dst_v)

    def _gather_start(ci, b):
        idx = src_v.at[pl.ds(ci * CHUNK, CHUNK)]

        @pl.when(cid == 0)
        def _():
            pltpu.make_async_copy(hlo_hbm.at[idx], rows_v.at[b], gsems[b]).start()

        @pl.when(cid == 1)
        def _():
            pltpu.make_async_copy(hhi_hbm.at[idx], rows_v.at[b], gsems[b]).start()

    def _gather_wait(ci, b):
        pltpu.make_async_copy(
            hlo_hbm.at[src_v.at[pl.ds(ci * CHUNK, CHUNK)]], rows_v.at[b],
            gsems[b]).wait()

    tail_idx = src_v.at[pl.ds(NFULL * CHUNK, TAIL)]

    @pl.when(cid == 0)
    def _():
        pltpu.make_async_copy(hlo_hbm.at[tail_idx],
                              rows_v.at[NBUF, pl.ds(0, TAIL)], gsems[NBUF]).start()

    @pl.when(cid == 1)
    def _():
        pltpu.make_async_copy(hhi_hbm.at[tail_idx],
                              rows_v.at[NBUF, pl.ds(0, TAIL)], gsems[NBUF]).start()

    for b in range(PREF):
        _gather_start(b, b)

    def _scatter_start(ci, b):
        pltpu.async_copy(rows_v.at[b],
                         acc_sh.at[dst_v.at[pl.ds(ci * CHUNK, CHUNK)]],
                         ssems[b], add=True)

    def _scatter_wait(ci, b):
        pltpu.make_async_copy(rows_v.at[b],
                              acc_sh.at[dst_v.at[pl.ds(ci * CHUNK, CHUNK)]],
                              ssems[b]).wait()

    def _chunk_body(ci, b):
        _gather_wait(ci, b)
        _scatter_start(ci, b)
        bn = (b + PREF) % NBUF

        @pl.when(ci + PREF < NFULL)
        def _():
            @pl.when(ci >= PREF)
            def _():
                _scatter_wait(ci - PREF, bn)

            _gather_start(ci + PREF, bn)

    def _step(k, _):
        i0 = k * NBUF
        for b in range(NBUF):
            _chunk_body(i0 + b, b)
        return 0

    lax.fori_loop(0, NFULL // NBUF, _step, 0)
    for ci in range((NFULL // NBUF) * NBUF, NFULL):
        _gather_wait(ci, ci % NBUF)
        _scatter_start(ci, ci % NBUF)

    for j in range(NBUF):
        ci = NFULL - NBUF + j
        _scatter_wait(ci, ci % NBUF)

    pltpu.make_async_copy(hlo_hbm.at[tail_idx],
                          rows_v.at[NBUF, pl.ds(0, TAIL)], gsems[NBUF]).wait()
    pltpu.sync_copy(rows_v.at[NBUF, pl.ds(0, TAIL)],
                    acc_sh.at[dst_v.at[pl.ds(NFULL * CHUNK, TAIL)]],
                    add=True)
    plsc.subcore_barrier()

    stripe = acc_sh.at[pl.ds(sid * ROWS_PER_TILE, ROWS_PER_TILE)]
    if interleave:
        out_hbm, = out_refs

        @pl.when(cid == 0)
        def _():
            pltpu.sync_copy(stripe,
                            out_hbm.at[pl.ds(sid * ROWS_PER_TILE, ROWS_PER_TILE),
                                       pl.ds(0, HALF)])

        @pl.when(cid == 1)
        def _():
            pltpu.sync_copy(stripe,
                            out_hbm.at[pl.ds(sid * ROWS_PER_TILE, ROWS_PER_TILE),
                                       pl.ds(HALF, HALF)])
    else:
        outlo_hbm, outhi_hbm = out_refs

        @pl.when(cid == 0)
        def _():
            pltpu.sync_copy(
                stripe, outlo_hbm.at[pl.ds(sid * ROWS_PER_TILE, ROWS_PER_TILE)])

        @pl.when(cid == 1)
        def _():
            pltpu.sync_copy(
                stripe, outhi_hbm.at[pl.ds(sid * ROWS_PER_TILE, ROWS_PER_TILE)])


_prop_out_split = [
    jax.ShapeDtypeStruct((N_NODES, HALF), jnp.float32),
    jax.ShapeDtypeStruct((N_NODES, HALF), jnp.float32),
]
_prop_out_full = [jax.ShapeDtypeStruct((N_NODES, OUT_DIM), jnp.float32)]
_prop_scratch = [
    pltpu.VMEM((EDGES_PER_T,), jnp.int32),
    pltpu.VMEM((EDGES_PER_T,), jnp.int32),
    pltpu.VMEM((NBUF + 1, CHUNK, HALF), jnp.float32),
    pltpu.VMEM_SHARED((N_NODES, HALF), jnp.float32),
    [pltpu.SemaphoreType.DMA] * (NBUF + 1),
    [pltpu.SemaphoreType.DMA] * NBUF,
]


@functools.partial(pl.kernel, out_type=_prop_out_split, mesh=_sc_mesh,
                   compiler_params=pltpu.CompilerParams(use_tc_tiling_on_sc=False),
                   scratch_types=_prop_scratch)
def _propagate_split(hlo, hhi, edge, outlo, outhi, src_v, dst_v, rows_v,
                     acc_sh, gsems, ssems):
    _propagate_body(False, hlo, hhi, edge, (outlo, outhi),
                    src_v, dst_v, rows_v, acc_sh, gsems, ssems)


@functools.partial(pl.kernel, out_type=_prop_out_full, mesh=_sc_mesh,
                   compiler_params=pltpu.CompilerParams(use_tc_tiling_on_sc=False),
                   scratch_types=_prop_scratch)
def _propagate_full(hlo, hhi, edge, out, src_v, dst_v, rows_v, acc_sh,
                    gsems, ssems):
    _propagate_body(True, hlo, hhi, edge, (out,),
                    src_v, dst_v, rows_v, acc_sh, gsems, ssems)


@jax.jit
def kernel(features, edge_index, W1, W2):
    h_lo, h_hi = _mlp(features, W1, W2)
    edges = edge_index.astype(jnp.int32).reshape(-1)
    for _ in range(NLAYERS - 1):
        h_lo, h_hi = _propagate_split(h_lo, h_hi, edges)
    out, = _propagate_full(h_lo, h_hi, edges)
    return out


# speedup vs baseline: 1.3310x; 1.1058x over previous
import functools

import jax
import jax.numpy as jnp
from jax import lax
from jax.experimental import pallas as pl
from jax.experimental.pallas import tpu as pltpu
from jax.experimental.pallas import tpu_sc as plsc

N_NODES = 10000
N_EDGES = 320000
IN_DIM = 128
HIDDEN_DIM = 128
OUT_DIM = 64
HALF = OUT_DIM // 2

NC = 2
NS = 16
EDGES_PER_T = N_EDGES // NS
CHUNK = 128
NFULL = EDGES_PER_T // CHUNK
TAIL = EDGES_PER_T - NFULL * CHUNK
ROWS_PER_TILE = N_NODES // NS
NBUF = 8
PREF = 4


def _mlp_body(x_ref, w1_ref, w2_ref, o_ref):
    h = jnp.dot(x_ref[...], w1_ref[...], preferred_element_type=jnp.float32)
    h = jnp.maximum(h, 0.0)
    o_ref[:, :OUT_DIM] = jnp.dot(h, w2_ref[...], preferred_element_type=jnp.float32)
    o_ref[:, OUT_DIM:] = jnp.zeros_like(o_ref[:, OUT_DIM:])


_MLP_GRID = 5
_mlp = pl.pallas_call(
    _mlp_body,
    grid=(_MLP_GRID,),
    in_specs=[
        pl.BlockSpec((N_NODES // _MLP_GRID, IN_DIM), lambda i: (i, 0)),
        pl.BlockSpec((IN_DIM, HIDDEN_DIM), lambda i: (0, 0)),
        pl.BlockSpec((HIDDEN_DIM, OUT_DIM), lambda i: (0, 0)),
    ],
    out_specs=pl.BlockSpec((N_NODES // _MLP_GRID, 2 * OUT_DIM), lambda i: (i, 0)),
    out_shape=jax.ShapeDtypeStruct((N_NODES, 2 * OUT_DIM), jnp.float32),
)


_sc_mesh = plsc.VectorSubcoreMesh(
    core_axis_name="c", subcore_axis_name="s", num_cores=NC, num_subcores=NS
)


@functools.partial(
    pl.kernel,
    out_type=[
        jax.ShapeDtypeStruct((N_NODES, OUT_DIM), jnp.float32),
        jax.ShapeDtypeStruct((N_NODES, HALF), jnp.float32),
        jax.ShapeDtypeStruct((N_NODES, HALF), jnp.float32),
    ],
    mesh=_sc_mesh,
    compiler_params=pltpu.CompilerParams(use_tc_tiling_on_sc=False),
    scratch_types=[
        pltpu.VMEM((EDGES_PER_T,), jnp.int32),
        pltpu.VMEM((EDGES_PER_T,), jnp.int32),
        pltpu.VMEM((NBUF + 1, CHUNK), jnp.int32),
        pltpu.VMEM((NBUF + 1, CHUNK, HALF), jnp.float32),
        pltpu.VMEM_SHARED((N_NODES, HALF), jnp.float32),
        [pltpu.SemaphoreType.DMA] * (NBUF + 1),
        [pltpu.SemaphoreType.DMA] * NBUF,
    ],
)
def _propagate2(h4_hbm, edge_hbm, out_hbm, lo_hbm, hi_hbm,
                src_v, dst_v, idx4_v, rows_v, acc_sh, gsems, ssems):
    cid = lax.axis_index("c")
    sid = lax.axis_index("s")
    zeros16 = jnp.zeros((16,), jnp.float32)

    def _zero_acc_stripe(nbufs):
        def _zero_row(r, _):
            for b in range(nbufs):
                for l in range(HALF // 16):
                    rows_v[b, r, pl.ds(l * 16, 16)] = zeros16
            return 0

        lax.fori_loop(0, CHUNK, _zero_row, 0)
        for k in range(5):
            pltpu.sync_copy(
                rows_v.at[k % nbufs, pl.ds(0, 125)],
                acc_sh.at[pl.ds(sid * ROWS_PER_TILE + k * 125, 125)],
            )

    _zero_acc_stripe(NBUF + 1)

    pltpu.sync_copy(edge_hbm.at[pl.ds(sid * EDGES_PER_T, EDGES_PER_T)], src_v)
    pltpu.sync_copy(
        edge_hbm.at[pl.ds(N_EDGES + sid * EDGES_PER_T, EDGES_PER_T)], dst_v)
    plsc.subcore_barrier()

    def _round(table_lo, table_hi, transform):
        def _prep_idx(ci, b, n):
            if transform:
                for l in range(n // 16):
                    s = src_v[pl.ds(ci * CHUNK + l * 16, 16)]
                    idx4_v[b, pl.ds(l * 16, 16)] = s * 4 + cid

        def _gidx(ci, b, n):
            if transform:
                return idx4_v.at[b, pl.ds(0, n)]
            return src_v.at[pl.ds(ci * CHUNK, n)]

        def _gather_start(ci, b, n=CHUNK, bufsl=None):
            _prep_idx(ci, b, n)
            dst = rows_v.at[b] if bufsl is None else bufsl
            if table_lo is table_hi:
                pltpu.make_async_copy(table_lo.at[_gidx(ci, b, n)], dst,
                                      gsems[b]).start()
            else:
                @pl.when(cid == 0)
                def _():
                    pltpu.make_async_copy(table_lo.at[_gidx(ci, b, n)], dst,
                                          gsems[b]).start()

                @pl.when(cid == 1)
                def _():
                    pltpu.make_async_copy(table_hi.at[_gidx(ci, b, n)], dst,
                                          gsems[b]).start()

        def _gather_wait(ci, b, n=CHUNK, bufsl=None):
            dst = rows_v.at[b] if bufsl is None else bufsl
            pltpu.make_async_copy(table_lo.at[_gidx(ci, b, n)], dst,
                                  gsems[b]).wait()

        def _scatter_start(ci, b):
            pltpu.async_copy(rows_v.at[b],
                             acc_sh.at[dst_v.at[pl.ds(ci * CHUNK, CHUNK)]],
                             ssems[b], add=True)

        def _scatter_wait(ci, b):
            pltpu.make_async_copy(rows_v.at[b],
                                  acc_sh.at[dst_v.at[pl.ds(ci * CHUNK, CHUNK)]],
                                  ssems[b]).wait()

        tail_buf = rows_v.at[NBUF, pl.ds(0, TAIL)]
        _gather_start(NFULL, NBUF, TAIL, tail_buf)
        for b in range(PREF):
            _gather_start(b, b)

        def _chunk_body(ci, b):
            _gather_wait(ci, b)
            _scatter_start(ci, b)
            bn = (b + PREF) % NBUF

            @pl.when(ci + PREF < NFULL)
            def _():
                @pl.when(ci >= PREF)
                def _():
                    _scatter_wait(ci - PREF, bn)

                _gather_start(ci + PREF, bn)

        def _step(k, _):
            i0 = k * NBUF
            for b in range(NBUF):
                _chunk_body(i0 + b, b)
            return 0

        lax.fori_loop(0, NFULL // NBUF, _step, 0)
        for ci in range((NFULL // NBUF) * NBUF, NFULL):
            _gather_wait(ci, ci % NBUF)
            _scatter_start(ci, ci % NBUF)

        for j in range(NBUF):
            ci = NFULL - NBUF + j
            _scatter_wait(ci, ci % NBUF)

        _gather_wait(NFULL, NBUF, TAIL, tail_buf)
        pltpu.sync_copy(tail_buf,
                        acc_sh.at[dst_v.at[pl.ds(NFULL * CHUNK, TAIL)]],
                        add=True)
        plsc.subcore_barrier()

    stripe = acc_sh.at[pl.ds(sid * ROWS_PER_TILE, ROWS_PER_TILE)]

    _round(h4_hbm, h4_hbm, True)

    @pl.when(cid == 0)
    def _():
        pltpu.sync_copy(stripe,
                        lo_hbm.at[pl.ds(sid * ROWS_PER_TILE, ROWS_PER_TILE)])

    @pl.when(cid == 1)
    def _():
        pltpu.sync_copy(stripe,
                        hi_hbm.at[pl.ds(sid * ROWS_PER_TILE, ROWS_PER_TILE)])

    _zero_acc_stripe(1)
    plsc.subcore_barrier()

    _round(lo_hbm, hi_hbm, False)

    @pl.when(cid == 0)
    def _():
        pltpu.sync_copy(stripe,
                        out_hbm.at[pl.ds(sid * ROWS_PER_TILE, ROWS_PER_TILE),
                                   pl.ds(0, HALF)])

    @pl.when(cid == 1)
    def _():
        pltpu.sync_copy(stripe,
                        out_hbm.at[pl.ds(sid * ROWS_PER_TILE, ROWS_PER_TILE),
                                   pl.ds(HALF, HALF)])


@jax.jit
def kernel(features, edge_index, W1, W2):
    h4 = _mlp(features, W1, W2).reshape(4 * N_NODES, HALF)
    edges = edge_index.astype(jnp.int32).reshape(-1)
    out, _, _ = _propagate2(h4, edges)
    return out


# speedup vs baseline: 1.4135x; 1.0620x over previous
import functools

import jax
import jax.numpy as jnp
from jax import lax
from jax.experimental import pallas as pl
from jax.experimental.pallas import tpu as pltpu
from jax.experimental.pallas import tpu_sc as plsc

N_NODES = 10000
N_EDGES = 320000
IN_DIM = 128
HIDDEN_DIM = 128
OUT_DIM = 64
HALF = OUT_DIM // 2

NC = 2
NS = 16
EDGES_PER_T = N_EDGES // NS
CHUNK = 128
NFULL = EDGES_PER_T // CHUNK
TAIL = EDGES_PER_T - NFULL * CHUNK
ROWS_PER_TILE = N_NODES // NS
NBUF = 12
PREF = 6


def _mlp_body(x_ref, w1_ref, w2_ref, o_ref):
    h = jnp.dot(x_ref[...], w1_ref[...], preferred_element_type=jnp.float32)
    h = jnp.maximum(h, 0.0)
    o_ref[:, :OUT_DIM] = jnp.dot(h, w2_ref[...], preferred_element_type=jnp.float32)
    o_ref[:, OUT_DIM:] = jnp.zeros_like(o_ref[:, OUT_DIM:])


_MLP_GRID = 5
_mlp = pl.pallas_call(
    _mlp_body,
    grid=(_MLP_GRID,),
    in_specs=[
        pl.BlockSpec((N_NODES // _MLP_GRID, IN_DIM), lambda i: (i, 0)),
        pl.BlockSpec((IN_DIM, HIDDEN_DIM), lambda i: (0, 0)),
        pl.BlockSpec((HIDDEN_DIM, OUT_DIM), lambda i: (0, 0)),
    ],
    out_specs=pl.BlockSpec((N_NODES // _MLP_GRID, 2 * OUT_DIM), lambda i: (i, 0)),
    out_shape=jax.ShapeDtypeStruct((N_NODES, 2 * OUT_DIM), jnp.float32),
)


_sc_mesh = plsc.VectorSubcoreMesh(
    core_axis_name="c", subcore_axis_name="s", num_cores=NC, num_subcores=NS
)


@functools.partial(
    pl.kernel,
    out_type=[
        jax.ShapeDtypeStruct((N_NODES, OUT_DIM), jnp.float32),
        jax.ShapeDtypeStruct((N_NODES, HALF), jnp.float32),
        jax.ShapeDtypeStruct((N_NODES, HALF), jnp.float32),
    ],
    mesh=_sc_mesh,
    compiler_params=pltpu.CompilerParams(use_tc_tiling_on_sc=False),
    scratch_types=[
        pltpu.VMEM((EDGES_PER_T,), jnp.int32),
        pltpu.VMEM((EDGES_PER_T,), jnp.int32),
        pltpu.VMEM((NBUF + 1, CHUNK), jnp.int32),
        pltpu.VMEM((NBUF + 1, CHUNK, HALF), jnp.float32),
        pltpu.VMEM_SHARED((N_NODES, HALF), jnp.float32),
        [pltpu.SemaphoreType.DMA] * (NBUF + 1),
        [pltpu.SemaphoreType.DMA] * NBUF,
    ],
)
def _propagate2(h4_hbm, edge_hbm, out_hbm, lo_hbm, hi_hbm,
                src_v, dst_v, idx4_v, rows_v, acc_sh, gsems, ssems):
    cid = lax.axis_index("c")
    sid = lax.axis_index("s")
    zeros16 = jnp.zeros((16,), jnp.float32)

    def _zero_acc_stripe(nbufs):
        def _zero_row(r, _):
            for b in range(nbufs):
                for l in range(HALF // 16):
                    rows_v[b, r, pl.ds(l * 16, 16)] = zeros16
            return 0

        lax.fori_loop(0, CHUNK, _zero_row, 0)
        for k in range(5):
            pltpu.sync_copy(
                rows_v.at[k % nbufs, pl.ds(0, 125)],
                acc_sh.at[pl.ds(sid * ROWS_PER_TILE + k * 125, 125)],
            )

    _zero_acc_stripe(NBUF + 1)

    pltpu.sync_copy(edge_hbm.at[pl.ds(sid * EDGES_PER_T, EDGES_PER_T)], src_v)
    pltpu.sync_copy(
        edge_hbm.at[pl.ds(N_EDGES + sid * EDGES_PER_T, EDGES_PER_T)], dst_v)
    plsc.subcore_barrier()

    def _round(table_lo, table_hi, transform):
        def _prep_idx(ci, b, n):
            if transform:
                for l in range(n // 16):
                    s = src_v[pl.ds(ci * CHUNK + l * 16, 16)]
                    idx4_v[b, pl.ds(l * 16, 16)] = s * 4 + cid

        def _gidx(ci, b, n):
            if transform:
                return idx4_v.at[b, pl.ds(0, n)]
            return src_v.at[pl.ds(ci * CHUNK, n)]

        def _gather_start(ci, b, n=CHUNK, bufsl=None):
            _prep_idx(ci, b, n)
            dst = rows_v.at[b] if bufsl is None else bufsl
            if table_lo is table_hi:
                pltpu.make_async_copy(table_lo.at[_gidx(ci, b, n)], dst,
                                      gsems[b]).start()
            else:
                @pl.when(cid == 0)
                def _():
                    pltpu.make_async_copy(table_lo.at[_gidx(ci, b, n)], dst,
                                          gsems[b]).start()

                @pl.when(cid == 1)
                def _():
                    pltpu.make_async_copy(table_hi.at[_gidx(ci, b, n)], dst,
                                          gsems[b]).start()

        def _gather_wait(ci, b, n=CHUNK, bufsl=None):
            dst = rows_v.at[b] if bufsl is None else bufsl
            pltpu.make_async_copy(table_lo.at[_gidx(ci, b, n)], dst,
                                  gsems[b]).wait()

        def _scatter_start(ci, b):
            pltpu.async_copy(rows_v.at[b],
                             acc_sh.at[dst_v.at[pl.ds(ci * CHUNK, CHUNK)]],
                             ssems[b], add=True)

        def _scatter_wait(ci, b):
            pltpu.make_async_copy(rows_v.at[b],
                                  acc_sh.at[dst_v.at[pl.ds(ci * CHUNK, CHUNK)]],
                                  ssems[b]).wait()

        tail_buf = rows_v.at[NBUF, pl.ds(0, TAIL)]
        _gather_start(NFULL, NBUF, TAIL, tail_buf)
        for b in range(PREF):
            _gather_start(b, b)

        def _chunk_body(ci, b):
            _gather_wait(ci, b)
            _scatter_start(ci, b)
            bn = (b + PREF) % NBUF

            @pl.when(ci + PREF < NFULL)
            def _():
                @pl.when(ci >= PREF)
                def _():
                    _scatter_wait(ci - PREF, bn)

                _gather_start(ci + PREF, bn)

        def _step(k, _):
            i0 = k * NBUF
            for b in range(NBUF):
                _chunk_body(i0 + b, b)
            return 0

        lax.fori_loop(0, NFULL // NBUF, _step, 0)
        for ci in range((NFULL // NBUF) * NBUF, NFULL):
            _gather_wait(ci, ci % NBUF)
            _scatter_start(ci, ci % NBUF)

        for j in range(NBUF):
            ci = NFULL - NBUF + j
            _scatter_wait(ci, ci % NBUF)

        _gather_wait(NFULL, NBUF, TAIL, tail_buf)
        pltpu.sync_copy(tail_buf,
                        acc_sh.at[dst_v.at[pl.ds(NFULL * CHUNK, TAIL)]],
                        add=True)
        plsc.subcore_barrier()

    stripe = acc_sh.at[pl.ds(sid * ROWS_PER_TILE, ROWS_PER_TILE)]

    _round(h4_hbm, h4_hbm, True)

    @pl.when(cid == 0)
    def _():
        pltpu.sync_copy(stripe,
                        lo_hbm.at[pl.ds(sid * ROWS_PER_TILE, ROWS_PER_TILE)])

    @pl.when(cid == 1)
    def _():
        pltpu.sync_copy(stripe,
                        hi_hbm.at[pl.ds(sid * ROWS_PER_TILE, ROWS_PER_TILE)])

    _zero_acc_stripe(1)
    plsc.subcore_barrier()

    _round(lo_hbm, hi_hbm, False)

    @pl.when(cid == 0)
    def _():
        pltpu.sync_copy(stripe,
                        out_hbm.at[pl.ds(sid * ROWS_PER_TILE, ROWS_PER_TILE),
                                   pl.ds(0, HALF)])

    @pl.when(cid == 1)
    def _():
        pltpu.sync_copy(stripe,
                        out_hbm.at[pl.ds(sid * ROWS_PER_TILE, ROWS_PER_TILE),
                                   pl.ds(HALF, HALF)])


@jax.jit
def kernel(features, edge_index, W1, W2):
    h4 = _mlp(features, W1, W2).reshape(4 * N_NODES, HALF)
    edges = edge_index.astype(jnp.int32).reshape(-1)
    out, _, _ = _propagate2(h4, edges)
    return out


# speedup vs baseline: 1.4270x; 1.0096x over previous
import functools

import jax
import jax.numpy as jnp
from jax import lax
from jax.experimental import pallas as pl
from jax.experimental.pallas import tpu as pltpu
from jax.experimental.pallas import tpu_sc as plsc

N_NODES = 10000
N_EDGES = 320000
IN_DIM = 128
HIDDEN_DIM = 128
OUT_DIM = 64
HALF = OUT_DIM // 2

NC = 2
NS = 16
EDGES_PER_T = N_EDGES // NS
CHUNK = 128
NFULL = EDGES_PER_T // CHUNK
TAIL = EDGES_PER_T - NFULL * CHUNK
ROWS_PER_TILE = N_NODES // NS
NBUF = 13
PREF = 7


def _mlp_body(x_ref, w1_ref, w2_ref, o_ref):
    h = jnp.dot(x_ref[...], w1_ref[...], preferred_element_type=jnp.float32)
    h = jnp.maximum(h, 0.0)
    o_ref[:, :OUT_DIM] = jnp.dot(h, w2_ref[...], preferred_element_type=jnp.float32)
    o_ref[:, OUT_DIM:] = jnp.zeros_like(o_ref[:, OUT_DIM:])


_MLP_GRID = 5
_mlp = pl.pallas_call(
    _mlp_body,
    grid=(_MLP_GRID,),
    in_specs=[
        pl.BlockSpec((N_NODES // _MLP_GRID, IN_DIM), lambda i: (i, 0)),
        pl.BlockSpec((IN_DIM, HIDDEN_DIM), lambda i: (0, 0)),
        pl.BlockSpec((HIDDEN_DIM, OUT_DIM), lambda i: (0, 0)),
    ],
    out_specs=pl.BlockSpec((N_NODES // _MLP_GRID, 2 * OUT_DIM), lambda i: (i, 0)),
    out_shape=jax.ShapeDtypeStruct((N_NODES, 2 * OUT_DIM), jnp.float32),
)


_sc_mesh = plsc.VectorSubcoreMesh(
    core_axis_name="c", subcore_axis_name="s", num_cores=NC, num_subcores=NS
)


@functools.partial(
    pl.kernel,
    out_type=[
        jax.ShapeDtypeStruct((N_NODES, OUT_DIM), jnp.float32),
        jax.ShapeDtypeStruct((N_NODES, HALF), jnp.float32),
        jax.ShapeDtypeStruct((N_NODES, HALF), jnp.float32),
    ],
    mesh=_sc_mesh,
    compiler_params=pltpu.CompilerParams(use_tc_tiling_on_sc=False),
    scratch_types=[
        pltpu.VMEM((EDGES_PER_T,), jnp.int32),
        pltpu.VMEM((EDGES_PER_T,), jnp.int32),
        pltpu.VMEM((NBUF + 1, CHUNK), jnp.int32),
        pltpu.VMEM((NBUF + 1, CHUNK, HALF), jnp.float32),
        pltpu.VMEM_SHARED((N_NODES, HALF), jnp.float32),
        [pltpu.SemaphoreType.DMA] * (NBUF + 1),
        [pltpu.SemaphoreType.DMA] * NBUF,
    ],
)
def _propagate2(h4_hbm, edge_hbm, out_hbm, lo_hbm, hi_hbm,
                src_v, dst_v, idx4_v, rows_v, acc_sh, gsems, ssems):
    cid = lax.axis_index("c")
    sid = lax.axis_index("s")
    zeros16 = jnp.zeros((16,), jnp.float32)

    def _zero_acc_stripe(nbufs):
        def _zero_row(r, _):
            for b in range(nbufs):
                for l in range(HALF // 16):
                    rows_v[b, r, pl.ds(l * 16, 16)] = zeros16
            return 0

        lax.fori_loop(0, CHUNK, _zero_row, 0)
        for k in range(5):
            pltpu.sync_copy(
                rows_v.at[k % nbufs, pl.ds(0, 125)],
                acc_sh.at[pl.ds(sid * ROWS_PER_TILE + k * 125, 125)],
            )

    _zero_acc_stripe(NBUF + 1)

    pltpu.sync_copy(edge_hbm.at[pl.ds(sid * EDGES_PER_T, EDGES_PER_T)], src_v)
    pltpu.sync_copy(
        edge_hbm.at[pl.ds(N_EDGES + sid * EDGES_PER_T, EDGES_PER_T)], dst_v)
    plsc.subcore_barrier()

    def _round(table_lo, table_hi, transform):
        def _prep_idx(ci, b, n):
            if transform:
                for l in range(n // 16):
                    s = src_v[pl.ds(ci * CHUNK + l * 16, 16)]
                    idx4_v[b, pl.ds(l * 16, 16)] = s * 4 + cid

        def _gidx(ci, b, n):
            if transform:
                return idx4_v.at[b, pl.ds(0, n)]
            return src_v.at[pl.ds(ci * CHUNK, n)]

        def _gather_start(ci, b, n=CHUNK, bufsl=None):
            _prep_idx(ci, b, n)
            dst = rows_v.at[b] if bufsl is None else bufsl
            if table_lo is table_hi:
                pltpu.make_async_copy(table_lo.at[_gidx(ci, b, n)], dst,
                                      gsems[b]).start()
            else:
                @pl.when(cid == 0)
                def _():
                    pltpu.make_async_copy(table_lo.at[_gidx(ci, b, n)], dst,
                                          gsems[b]).start()

                @pl.when(cid == 1)
                def _():
                    pltpu.make_async_copy(table_hi.at[_gidx(ci, b, n)], dst,
                                          gsems[b]).start()

        def _gather_wait(ci, b, n=CHUNK, bufsl=None):
            dst = rows_v.at[b] if bufsl is None else bufsl
            pltpu.make_async_copy(table_lo.at[_gidx(ci, b, n)], dst,
                                  gsems[b]).wait()

        def _scatter_start(ci, b):
            pltpu.async_copy(rows_v.at[b],
                             acc_sh.at[dst_v.at[pl.ds(ci * CHUNK, CHUNK)]],
                             ssems[b], add=True)

        def _scatter_wait(ci, b):
            pltpu.make_async_copy(rows_v.at[b],
                                  acc_sh.at[dst_v.at[pl.ds(ci * CHUNK, CHUNK)]],
                                  ssems[b]).wait()

        tail_buf = rows_v.at[NBUF, pl.ds(0, TAIL)]
        _gather_start(NFULL, NBUF, TAIL, tail_buf)
        for b in range(PREF):
            _gather_start(b, b)

        def _chunk_body(ci, b):
            _gather_wait(ci, b)
            _scatter_start(ci, b)
            bn = (b + PREF) % NBUF

            @pl.when(ci + PREF < NFULL)
            def _():
                @pl.when(ci >= PREF)
                def _():
                    _scatter_wait(ci - PREF, bn)

                _gather_start(ci + PREF, bn)

        def _step(k, _):
            i0 = k * NBUF
            for b in range(NBUF):
                _chunk_body(i0 + b, b)
            return 0

        lax.fori_loop(0, NFULL // NBUF, _step, 0)
        for ci in range((NFULL // NBUF) * NBUF, NFULL):
            _gather_wait(ci, ci % NBUF)
            _scatter_start(ci, ci % NBUF)

        for j in range(NBUF):
            ci = NFULL - NBUF + j
            _scatter_wait(ci, ci % NBUF)

        _gather_wait(NFULL, NBUF, TAIL, tail_buf)
        pltpu.sync_copy(tail_buf,
                        acc_sh.at[dst_v.at[pl.ds(NFULL * CHUNK, TAIL)]],
                        add=True)
        plsc.subcore_barrier()

    stripe = acc_sh.at[pl.ds(sid * ROWS_PER_TILE, ROWS_PER_TILE)]

    _round(h4_hbm, h4_hbm, True)

    @pl.when(cid == 0)
    def _():
        pltpu.sync_copy(stripe,
                        lo_hbm.at[pl.ds(sid * ROWS_PER_TILE, ROWS_PER_TILE)])

    @pl.when(cid == 1)
    def _():
        pltpu.sync_copy(stripe,
                        hi_hbm.at[pl.ds(sid * ROWS_PER_TILE, ROWS_PER_TILE)])

    _zero_acc_stripe(1)
    plsc.subcore_barrier()

    _round(lo_hbm, hi_hbm, False)

    @pl.when(cid == 0)
    def _():
        pltpu.sync_copy(stripe,
                        out_hbm.at[pl.ds(sid * ROWS_PER_TILE, ROWS_PER_TILE),
                                   pl.ds(0, HALF)])

    @pl.when(cid == 1)
    def _():
        pltpu.sync_copy(stripe,
                        out_hbm.at[pl.ds(sid * ROWS_PER_TILE, ROWS_PER_TILE),
                                   pl.ds(HALF, HALF)])


@jax.jit
def kernel(features, edge_index, W1, W2):
    h4 = _mlp(features, W1, W2).reshape(4 * N_NODES, HALF)
    edges = edge_index.astype(jnp.int32).reshape(-1)
    out, _, _ = _propagate2(h4, edges)
    return out
